# baseline (speedup 1.0000x reference)
import functools

import jax
import jax.numpy as jnp
from jax import lax
from jax.experimental import pallas as pl
from jax.experimental.pallas import tpu as pltpu
from jax.experimental.pallas import tpu_sc as plsc

N_NODES = 10000
N_EDGES = 320000
H = 128

NUM_CORES = 2
NUM_SUBCORES = 16
NW = NUM_CORES * NUM_SUBCORES
CHUNK = 128
NBUF = 2
SB = 16
NSB = 10
HH = 64
CPW = SB * NSB
E_PAD = NUM_SUBCORES * CPW * CHUNK
ROWS_PER_TILE = 632
AGG_ROWS = NUM_SUBCORES * ROWS_PER_TILE
N_DEG = 10240

ROW_BLK = 400
N_BLKS = N_NODES // ROW_BLK
ROW_BLK_C = 512
N_BLKS_C = -(-N_NODES // ROW_BLK_C)


def _dense_body(x_ref, w1_ref, b1_ref, wl_ref, bl_ref, wr_ref, g_ref, r_ref):
    h = lax.dot_general(x_ref[...], w1_ref[...], (((1,), (1,)), ((), ())),
                        preferred_element_type=jnp.float32)
    h = jnp.maximum(h + b1_ref[...], 0.0)
    g_ref[...] = lax.dot_general(h, wl_ref[...], (((1,), (1,)), ((), ())),
                                 preferred_element_type=jnp.float32)
    r_ref[...] = lax.dot_general(h, wr_ref[...], (((1,), (1,)), ((), ())),
                                 preferred_element_type=jnp.float32) + bl_ref[...]


def _combine_body(p_ref, d_ref, r_ref, o_ref):
    i = pl.program_id(0)
    s = jnp.concatenate([p_ref[0], p_ref[1]], axis=1)
    dblk = d_ref[:, pl.ds(i * ROW_BLK_C, ROW_BLK_C)]
    deg = jnp.sum(dblk, axis=0).reshape(ROW_BLK_C, 1)
    o_ref[...] = s / jnp.clip(deg, 1.0, None) + r_ref[...]


def _sc_aggregate_body(g_hbm, src_hbm, dst_hbm, zeros_hbm, zdeg_hbm,
                       parts_hbm, degp_hbm,
                       src_v, dst_v, rows_v, deg_v, g_s, agg_s, *sems):
    cid = lax.axis_index("c")
    sid = lax.axis_index("s")
    wid = cid * NUM_SUBCORES + sid
    base = sid * ROWS_PER_TILE

    pltpu.sync_copy(g_hbm.at[pl.ds(base, ROWS_PER_TILE), pl.ds(cid * HH, HH)],
                    g_s.at[pl.ds(base, ROWS_PER_TILE)])
    pltpu.sync_copy(zeros_hbm, agg_s.at[pl.ds(base, ROWS_PER_TILE)])
    pltpu.sync_copy(zdeg_hbm, deg_v)
    plsc.subcore_barrier()

    ones16 = jnp.ones((16,), jnp.float32)

    def deg_update(k):
        for j in range(CHUNK // 16):
            iv = dst_v[k, pl.ds(j * 16, 16)]
            plsc.addupdate_scatter(deg_v, [iv], ones16)

    def stage(s, carry):
        pltpu.sync_copy(src_hbm.at[sid, pl.ds(s * SB, SB)], src_v)
        pltpu.sync_copy(dst_hbm.at[sid, pl.ds(s * SB, SB)], dst_v)
        for l in range(NBUF):
            pltpu.async_copy(g_s.at[src_v.at[l]], rows_v.at[l], sems[l])

        def body(q, c):
            for l in range(NBUF):
                k = NBUF * q + l
                pltpu.make_async_copy(g_s.at[src_v.at[k]],
                                      rows_v.at[l], sems[l]).wait()
                pltpu.sync_copy(rows_v.at[l], agg_s.at[dst_v.at[k]], add=True)

                @pl.when(cid == 0)
                def _():
                    deg_update(k)

                @pl.when(k + NBUF < SB)
                def _():
                    pltpu.async_copy(g_s.at[src_v.at[k + NBUF]],
                                     rows_v.at[l], sems[l])
            return c

        return lax.fori_loop(0, SB // NBUF, body, carry)

    lax.fori_loop(0, NSB, stage, 0)

    plsc.subcore_barrier()
    pltpu.sync_copy(agg_s.at[pl.ds(base, ROWS_PER_TILE)],
                    parts_hbm.at[cid, pl.ds(base, ROWS_PER_TILE)])
    pltpu.sync_copy(deg_v, degp_hbm.at[wid])


_sc_aggregate = functools.partial(
    pl.kernel,
    out_type=(jax.ShapeDtypeStruct((NUM_CORES, AGG_ROWS, HH), jnp.float32),
              jax.ShapeDtypeStruct((NW, N_DEG), jnp.float32)),
    mesh=plsc.VectorSubcoreMesh(core_axis_name="c", subcore_axis_name="s"),
    scratch_types=[
        pltpu.VMEM((SB, CHUNK), jnp.int32),
        pltpu.VMEM((SB, CHUNK), jnp.int32),
        pltpu.VMEM((NBUF, CHUNK, HH), jnp.float32),
        pltpu.VMEM((N_DEG,), jnp.float32),
        pltpu.VMEM_SHARED((AGG_ROWS, HH), jnp.float32),
        pltpu.VMEM_SHARED((AGG_ROWS, HH), jnp.float32),
    ] + [pltpu.SemaphoreType.DMA] * NBUF,
    compiler_params=pltpu.CompilerParams(needs_layout_passes=False,
                                         use_tc_tiling_on_sc=False),
)(_sc_aggregate_body)


def kernel(x, edge_index, W1, b1, Wl, bl, Wr):
    src = edge_index[0].astype(jnp.int32)
    dst = edge_index[1].astype(jnp.int32)
    pad = E_PAD - N_EDGES
    src_p = jnp.concatenate([src, jnp.zeros((pad,), jnp.int32)])
    dst_p = jnp.concatenate([dst, jnp.full((pad,), N_NODES, jnp.int32)])
    src_p = src_p.reshape(NUM_SUBCORES, CPW, CHUNK)
    dst_p = dst_p.reshape(NUM_SUBCORES, CPW, CHUNK)
    zeros = jnp.zeros((ROWS_PER_TILE, HH), jnp.float32)
    zdeg = jnp.zeros((N_DEG,), jnp.float32)

    g, r = pl.pallas_call(
        _dense_body,
        grid=(N_BLKS,),
        in_specs=[
            pl.BlockSpec((ROW_BLK, H), lambda i: (i, 0)),
            pl.BlockSpec((H, H), lambda i: (0, 0)),
            pl.BlockSpec((1, H), lambda i: (0, 0)),
            pl.BlockSpec((H, H), lambda i: (0, 0)),
            pl.BlockSpec((1, H), lambda i: (0, 0)),
            pl.BlockSpec((H, H), lambda i: (0, 0)),
        ],
        out_specs=[
            pl.BlockSpec((ROW_BLK, H), lambda i: (i, 0)),
            pl.BlockSpec((ROW_BLK, H), lambda i: (i, 0)),
        ],
        out_shape=[
            jax.ShapeDtypeStruct((AGG_ROWS, H), jnp.float32),
            jax.ShapeDtypeStruct((N_NODES, H), jnp.float32),
        ],
    )(x, W1, b1.reshape(1, H), Wl, bl.reshape(1, H), Wr)

    parts, degp = _sc_aggregate(g, src_p, dst_p, zeros, zdeg)

    out = pl.pallas_call(
        _combine_body,
        grid=(N_BLKS_C,),
        in_specs=[
            pl.BlockSpec((NUM_CORES, ROW_BLK_C, HH), lambda i: (0, i, 0)),
            pl.BlockSpec((NW, N_DEG), lambda i: (0, 0)),
            pl.BlockSpec((ROW_BLK_C, H), lambda i: (i, 0)),
        ],
        out_specs=pl.BlockSpec((ROW_BLK_C, H), lambda i: (i, 0)),
        out_shape=jax.ShapeDtypeStruct((N_NODES, H), jnp.float32),
    )(parts, degp, r)

    return out


# speedup vs baseline: 1.0043x; 1.0043x over previous
import functools

import jax
import jax.numpy as jnp
from jax import lax
from jax.experimental import pallas as pl
from jax.experimental.pallas import tpu as pltpu
from jax.experimental.pallas import tpu_sc as plsc

N_NODES = 10000
N_EDGES = 320000
H = 128

NUM_CORES = 2
NUM_SUBCORES = 16
NW = NUM_CORES * NUM_SUBCORES
CHUNK = 128
NBUF = 2
SB = 16
NSB = 10
HH = 64
CPW = SB * NSB
E_PAD = NUM_SUBCORES * CPW * CHUNK
ROWS_PER_TILE = 632
AGG_ROWS = NUM_SUBCORES * ROWS_PER_TILE
N_DEG = 10240

ROW_BLK = 400
N_BLKS = N_NODES // ROW_BLK
ROW_BLK_C = 512
N_BLKS_C = -(-N_NODES // ROW_BLK_C)


def _dense_body(x_ref, w1_ref, b1_ref, wl_ref, bl_ref, wr_ref, g_ref, r_ref):
    h = lax.dot_general(x_ref[...], w1_ref[...], (((1,), (1,)), ((), ())),
                        preferred_element_type=jnp.float32)
    h = jnp.maximum(h + b1_ref[...], 0.0)
    g_ref[...] = lax.dot_general(h, wl_ref[...], (((1,), (1,)), ((), ())),
                                 preferred_element_type=jnp.float32)
    r_ref[...] = lax.dot_general(h, wr_ref[...], (((1,), (1,)), ((), ())),
                                 preferred_element_type=jnp.float32) + bl_ref[...]


def _combine_body(p_ref, d_ref, r_ref, o_ref):
    i = pl.program_id(0)
    s = jnp.concatenate([p_ref[0], p_ref[1]], axis=1)
    dblk = d_ref[:, pl.ds(i * ROW_BLK_C, ROW_BLK_C)]
    deg = jnp.sum(dblk, axis=0).reshape(ROW_BLK_C, 1)
    o_ref[...] = s / jnp.clip(deg, 1.0, None) + r_ref[...]


def _sc_aggregate_body(g_hbm, src_hbm, dst_hbm, zeros_hbm, zdeg_hbm,
                       parts_hbm, degp_hbm,
                       src_v, dst_v, rows_v, deg_v, g_s, agg_s, *sems):
    cid = lax.axis_index("c")
    sid = lax.axis_index("s")
    wid = cid * NUM_SUBCORES + sid
    base = sid * ROWS_PER_TILE

    pltpu.sync_copy(g_hbm.at[pl.ds(base, ROWS_PER_TILE), pl.ds(cid * HH, HH)],
                    g_s.at[pl.ds(base, ROWS_PER_TILE)])
    pltpu.sync_copy(zeros_hbm, agg_s.at[pl.ds(base, ROWS_PER_TILE)])
    pltpu.sync_copy(zdeg_hbm, deg_v)
    plsc.subcore_barrier()

    ones16 = jnp.ones((16,), jnp.float32)

    def deg_update(k):
        for j in range(CHUNK // 16):
            iv = dst_v[k, pl.ds(j * 16, 16)]
            plsc.addupdate_scatter(deg_v, [iv], ones16)

    def stage(s, carry):
        pltpu.sync_copy(src_hbm.at[sid, pl.ds(s * SB, SB)], src_v)
        pltpu.sync_copy(dst_hbm.at[sid, pl.ds(s * SB, SB)], dst_v)
        do_deg = lax.rem(s, 2) == cid
        for l in range(NBUF):
            pltpu.async_copy(g_s.at[src_v.at[l]], rows_v.at[l], sems[l])

        def body(q, c):
            for l in range(NBUF):
                k = NBUF * q + l
                pltpu.make_async_copy(g_s.at[src_v.at[k]],
                                      rows_v.at[l], sems[l]).wait()
                pltpu.sync_copy(rows_v.at[l], agg_s.at[dst_v.at[k]], add=True)

                @pl.when(do_deg)
                def _():
                    deg_update(k)

                @pl.when(k + NBUF < SB)
                def _():
                    pltpu.async_copy(g_s.at[src_v.at[k + NBUF]],
                                     rows_v.at[l], sems[l])
            return c

        return lax.fori_loop(0, SB // NBUF, body, carry)

    lax.fori_loop(0, NSB, stage, 0)

    plsc.subcore_barrier()
    pltpu.sync_copy(agg_s.at[pl.ds(base, ROWS_PER_TILE)],
                    parts_hbm.at[cid, pl.ds(base, ROWS_PER_TILE)])
    pltpu.sync_copy(deg_v, degp_hbm.at[wid])


_sc_aggregate = functools.partial(
    pl.kernel,
    out_type=(jax.ShapeDtypeStruct((NUM_CORES, AGG_ROWS, HH), jnp.float32),
              jax.ShapeDtypeStruct((NW, N_DEG), jnp.float32)),
    mesh=plsc.VectorSubcoreMesh(core_axis_name="c", subcore_axis_name="s"),
    scratch_types=[
        pltpu.VMEM((SB, CHUNK), jnp.int32),
        pltpu.VMEM((SB, CHUNK), jnp.int32),
        pltpu.VMEM((NBUF, CHUNK, HH), jnp.float32),
        pltpu.VMEM((N_DEG,), jnp.float32),
        pltpu.VMEM_SHARED((AGG_ROWS, HH), jnp.float32),
        pltpu.VMEM_SHARED((AGG_ROWS, HH), jnp.float32),
    ] + [pltpu.SemaphoreType.DMA] * NBUF,
    compiler_params=pltpu.CompilerParams(needs_layout_passes=False,
                                         use_tc_tiling_on_sc=False),
)(_sc_aggregate_body)


def kernel(x, edge_index, W1, b1, Wl, bl, Wr):
    src = edge_index[0].astype(jnp.int32)
    dst = edge_index[1].astype(jnp.int32)
    pad = E_PAD - N_EDGES
    src_p = jnp.concatenate([src, jnp.zeros((pad,), jnp.int32)])
    dst_p = jnp.concatenate([dst, jnp.full((pad,), N_NODES, jnp.int32)])
    src_p = src_p.reshape(NUM_SUBCORES, CPW, CHUNK)
    dst_p = dst_p.reshape(NUM_SUBCORES, CPW, CHUNK)
    zeros = jnp.zeros((ROWS_PER_TILE, HH), jnp.float32)
    zdeg = jnp.zeros((N_DEG,), jnp.float32)

    g, r = pl.pallas_call(
        _dense_body,
        grid=(N_BLKS,),
        in_specs=[
            pl.BlockSpec((ROW_BLK, H), lambda i: (i, 0)),
            pl.BlockSpec((H, H), lambda i: (0, 0)),
            pl.BlockSpec((1, H), lambda i: (0, 0)),
            pl.BlockSpec((H, H), lambda i: (0, 0)),
            pl.BlockSpec((1, H), lambda i: (0, 0)),
            pl.BlockSpec((H, H), lambda i: (0, 0)),
        ],
        out_specs=[
            pl.BlockSpec((ROW_BLK, H), lambda i: (i, 0)),
            pl.BlockSpec((ROW_BLK, H), lambda i: (i, 0)),
        ],
        out_shape=[
            jax.ShapeDtypeStruct((AGG_ROWS, H), jnp.float32),
            jax.ShapeDtypeStruct((N_NODES, H), jnp.float32),
        ],
    )(x, W1, b1.reshape(1, H), Wl, bl.reshape(1, H), Wr)

    parts, degp = _sc_aggregate(g, src_p, dst_p, zeros, zdeg)

    out = pl.pallas_call(
        _combine_body,
        grid=(N_BLKS_C,),
        in_specs=[
            pl.BlockSpec((NUM_CORES, ROW_BLK_C, HH), lambda i: (0, i, 0)),
            pl.BlockSpec((NW, N_DEG), lambda i: (0, 0)),
            pl.BlockSpec((ROW_BLK_C, H), lambda i: (i, 0)),
        ],
        out_specs=pl.BlockSpec((ROW_BLK_C, H), lambda i: (i, 0)),
        out_shape=jax.ShapeDtypeStruct((N_NODES, H), jnp.float32),
    )(parts, degp, r)

    return out
